# Initial kernel scaffold; baseline (speedup 1.0000x reference)
import functools

import jax
import jax.numpy as jnp
from jax import lax
from jax.experimental import pallas as pl
from jax.experimental.pallas import tpu as pltpu
from jax.experimental.pallas import tpu_sc as plsc

N = 10000
E = 320000
D = 128

NW = 32
R = 320
NP = NW * R
CS = 4000
NCHUNK = E // CS
NGROUP = CS // 16
PCAP = 128
FLUSH = 112
LANES = 16
DC = D // LANES

NEG = jnp.float32(-jnp.inf)


def _sc_body(x_hbm, src_hbm, dst_hbm, sum_hbm, cnt_hbm, max_hbm,
             accs, accm, cnt, dstb, srcb, psrc, pdst, rows, sem):
    wid = lax.axis_index("s") * 2 + lax.axis_index("c")
    lo = wid * R
    hi = lo + R

    zs = jnp.zeros((LANES,), jnp.float32)
    negs = jnp.full((LANES,), NEG, jnp.float32)
    zi = jnp.zeros((LANES,), jnp.int32)

    def init_acc(i, _):
        accs[pl.ds(i * LANES, LANES)] = zs
        accm[pl.ds(i * LANES, LANES)] = negs
        return 0
    lax.fori_loop(0, R * DC, init_acc, 0)

    def init_cnt(i, _):
        cnt[pl.ds(i * LANES, LANES)] = zs
        return 0
    lax.fori_loop(0, R // LANES, init_cnt, 0)

    def init_pend(i, _):
        psrc[pl.ds(i * LANES, LANES)] = zi
        pdst[pl.ds(i * LANES, LANES)] = zi
        return 0
    lax.fori_loop(0, PCAP // LANES, init_pend, 0)

    def do_flush(pos):
        pltpu.async_copy(x_hbm.at[psrc], rows, sem).wait()

        def acc_one(e, _):
            d = pdst[e]
            off = d * D
            for c in range(DC):
                r = rows[e, pl.ds(c * LANES, LANES)]
                s = accs[pl.ds(off + c * LANES, LANES)]
                accs[pl.ds(off + c * LANES, LANES)] = s + r
                m = accm[pl.ds(off + c * LANES, LANES)]
                accm[pl.ds(off + c * LANES, LANES)] = jnp.maximum(m, r)
            cnt[d] = cnt[d] + 1.0
            return 0
        lax.fori_loop(0, pos, acc_one, 0)

    def scan_group(g, pos):
        base = g * LANES
        vd = dstb[pl.ds(base, LANES)]
        vs = srcb[pl.ds(base, LANES)]
        msk = (vd >= lo) & (vd < hi)
        plsc.store_compressed(psrc.at[pl.ds(pos, LANES)], vs, msk)
        plsc.store_compressed(pdst.at[pl.ds(pos, LANES)], vd - lo, msk)
        pos = pos + jnp.sum(msk.astype(jnp.int32))

        @pl.when(pos >= FLUSH)
        def _():
            do_flush(pos)
        return jnp.where(pos >= FLUSH, 0, pos)

    def chunk_body(ci, pos):
        pltpu.sync_copy(dst_hbm.at[pl.ds(ci * CS, CS)], dstb)
        pltpu.sync_copy(src_hbm.at[pl.ds(ci * CS, CS)], srcb)
        return lax.fori_loop(0, NGROUP, scan_group, pos)

    pos = lax.fori_loop(0, NCHUNK, chunk_body, jnp.int32(0))

    @pl.when(pos > 0)
    def _():
        do_flush(pos)

    pltpu.sync_copy(accs, sum_hbm.at[pl.ds(lo * D, R * D)])
    pltpu.sync_copy(accm, max_hbm.at[pl.ds(lo * D, R * D)])
    pltpu.sync_copy(cnt, cnt_hbm.at[pl.ds(lo, R)])


@jax.jit
def _sc_aggregate(x, src, dst):
    mesh = plsc.VectorSubcoreMesh(core_axis_name="c", subcore_axis_name="s")
    f = pl.kernel(
        _sc_body,
        mesh=mesh,
        out_type=(
            jax.ShapeDtypeStruct((NP * D,), jnp.float32),
            jax.ShapeDtypeStruct((NP,), jnp.float32),
            jax.ShapeDtypeStruct((NP * D,), jnp.float32),
        ),
        scratch_types=[
            pltpu.VMEM((R * D,), jnp.float32),
            pltpu.VMEM((R * D,), jnp.float32),
            pltpu.VMEM((R,), jnp.float32),
            pltpu.VMEM((CS,), jnp.int32),
            pltpu.VMEM((CS,), jnp.int32),
            pltpu.VMEM((PCAP,), jnp.int32),
            pltpu.VMEM((PCAP,), jnp.int32),
            pltpu.VMEM((PCAP, D), jnp.float32),
            pltpu.SemaphoreType.DMA,
        ],
    )
    return f(x, src, dst)


def _tc_body(x_ref, s_ref, c_ref, m_ref, w1_ref, w2_ref, w3_ref, b_ref,
             o_ref):
    c = c_ref[...]
    mean = s_ref[...] / jnp.maximum(c, 1.0)
    mx = jnp.where(c > 0.0, m_ref[...], 0.0)
    acc = jnp.dot(x_ref[...], w1_ref[...], preferred_element_type=jnp.float32)
    acc += jnp.dot(mean, w2_ref[...], preferred_element_type=jnp.float32)
    acc += jnp.dot(mx, w3_ref[...], preferred_element_type=jnp.float32)
    o_ref[...] = jnp.maximum(acc + b_ref[...], 0.0)


@jax.jit
def _tc_apply(x, s, c, m, w1, w2, w3, b):
    BN = 400
    grid = (N // BN,)
    row_spec = pl.BlockSpec((BN, D), lambda i: (i, 0))
    full = lambda shape: pl.BlockSpec(shape, lambda i: (0, 0))
    return pl.pallas_call(
        _tc_body,
        grid=grid,
        in_specs=[row_spec,
                  row_spec,
                  pl.BlockSpec((BN, 1), lambda i: (i, 0)),
                  row_spec,
                  full((D, D)), full((D, D)), full((D, D)),
                  full((1, D))],
        out_specs=row_spec,
        out_shape=jax.ShapeDtypeStruct((N, D), jnp.float32),
    )(x, s, c, m, w1, w2, w3, b)


def kernel(x, edge_index, W, b):
    src = edge_index[0]
    dst = edge_index[1]
    s_flat, cnt, m_flat = _sc_aggregate(x, src, dst)
    s = s_flat.reshape(NP, D)[:N]
    m = m_flat.reshape(NP, D)[:N]
    c = cnt[:N, None]
    w1 = W[:, :D].T
    w2 = W[:, D:2 * D].T
    w3 = W[:, 2 * D:].T
    return _tc_apply(x, s, c, m, w1, w2, w3, b[None, :])


# speedup vs baseline: 3.1002x; 3.1002x over previous
import functools

import jax
import jax.numpy as jnp
from jax import lax
from jax.experimental import pallas as pl
from jax.experimental.pallas import tpu as pltpu
from jax.experimental.pallas import tpu_sc as plsc

N = 10000
E = 320000
D = 128

NW = 32
R = 320
RA = R + 16
DUMP = R
NP = NW * R
CS = 3200
NCHUNK = E // CS
BG = 8
NBATCH = CS // (16 * BG)
PCAP = 128
PDCAP = 144
LANES = 16
DC = D // LANES

NEG = float("-inf")


def _sc_body(x_hbm, src_hbm, dst_hbm, sum_hbm, cnt_hbm, max_hbm,
             accs, accm, cnt, dstb, srcb, psrc, pdst, rows, sem, sem2):
    wid = lax.axis_index("s") * 2 + lax.axis_index("c")
    lo = wid * R
    hi = lo + R

    zs = jnp.zeros((LANES,), jnp.float32)
    negs = jnp.full((LANES,), NEG, jnp.float32)
    zi = jnp.zeros((LANES,), jnp.int32)

    def init_acc(i, _):
        accs[pl.ds(i * LANES, LANES)] = zs
        accm[pl.ds(i * LANES, LANES)] = negs
        return 0
    lax.fori_loop(0, RA * DC, init_acc, 0)

    def init_cnt(i, _):
        cnt[pl.ds(i * LANES, LANES)] = zs
        return 0
    lax.fori_loop(0, RA // LANES, init_cnt, 0)

    def init_pend(i, _):
        pdst[pl.ds(i * LANES, LANES)] = zi
        return 0
    lax.fori_loop(0, PDCAP // LANES, init_pend, 0)

    def init_psrc(i, _):
        psrc[pl.ds(i * LANES, LANES)] = zi
        return 0
    lax.fori_loop(0, PCAP // LANES, init_psrc, 0)

    lanes_iota = lax.iota(jnp.int32, LANES)
    one = jnp.float32(1.0)
    zero = jnp.float32(0.0)

    def do_flush(pos):
        pltpu.async_copy(x_hbm.at[psrc], rows, sem).wait()
        blk = pos >> 4
        rem = pos & (LANES - 1)

        @pl.when(rem > 0)
        def _():
            vdt = pdst[pl.ds(blk * LANES, LANES)]
            pdst[pl.ds(blk * LANES, LANES)] = jnp.where(
                lanes_iota >= rem, jnp.int32(DUMP), vdt)

        nb = blk + jnp.where(rem > 0, 1, 0)

        def acc_grp(g, _):
            base = g * LANES
            dv = pdst[pl.ds(base, LANES)]
            for i in range(LANES):
                d = dv[i]
                off = d * D
                e = base + i
                for c in range(DC):
                    r = rows[e, pl.ds(c * LANES, LANES)]
                    plsc.addupdate(accs.at[pl.ds(off + c * LANES, LANES)], r)
                    m = accm[pl.ds(off + c * LANES, LANES)]
                    accm[pl.ds(off + c * LANES, LANES)] = jnp.maximum(m, r)
                onehot = jnp.where(lanes_iota == (d & (LANES - 1)), one, zero)
                plsc.addupdate(cnt.at[pl.ds((d >> 4) * LANES, LANES)], onehot)
            return 0
        lax.fori_loop(0, nb, acc_grp, 0)

    one_i = jnp.int32(1)
    zero_i = jnp.int32(0)

    def chunk_body(ci, pos):
        hoff = (ci & 1) * CS
        pltpu.make_async_copy(dst_hbm.at[pl.ds(ci * CS, CS)],
                              dstb.at[pl.ds(hoff, CS)], sem2).wait()
        pltpu.make_async_copy(src_hbm.at[pl.ds(ci * CS, CS)],
                              srcb.at[pl.ds(hoff, CS)], sem2).wait()

        @pl.when(ci + 1 < NCHUNK)
        def _():
            nhoff = CS - hoff
            pltpu.async_copy(dst_hbm.at[pl.ds((ci + 1) * CS, CS)],
                             dstb.at[pl.ds(nhoff, CS)], sem2)
            pltpu.async_copy(src_hbm.at[pl.ds((ci + 1) * CS, CS)],
                             srcb.at[pl.ds(nhoff, CS)], sem2)

        def scan_batch(bi, pos):
            base = hoff + bi * (BG * LANES)
            vds = [dstb[pl.ds(base + k * LANES, LANES)] for k in range(BG)]
            vss = [srcb[pl.ds(base + k * LANES, LANES)] for k in range(BG)]
            msks = [(vd >= lo) & (vd < hi) for vd in vds]
            cums = [plsc.cumsum(jnp.where(m, one_i, zero_i)) for m in msks]
            cnts = [c[LANES - 1] for c in cums]
            total = cnts[0]
            for k in range(1, BG):
                total = total + cnts[k]

            @pl.when(pos + total > PCAP)
            def _():
                do_flush(pos)

            b = jnp.where(pos + total > PCAP, 0, pos)
            newpos = b + total
            for k in range(BG):
                idxv = b + cums[k] - 1
                plsc.store_scatter(psrc, [idxv], vss[k], mask=msks[k])
                plsc.store_scatter(pdst, [idxv], vds[k] - lo, mask=msks[k])
                b = b + cnts[k]
            return newpos

        return lax.fori_loop(0, NBATCH, scan_batch, pos)

    pltpu.async_copy(dst_hbm.at[pl.ds(0, CS)], dstb.at[pl.ds(0, CS)], sem2)
    pltpu.async_copy(src_hbm.at[pl.ds(0, CS)], srcb.at[pl.ds(0, CS)], sem2)
    pos = lax.fori_loop(0, NCHUNK, chunk_body, jnp.int32(0))

    @pl.when(pos > 0)
    def _():
        do_flush(pos)

    pltpu.sync_copy(accs.at[pl.ds(0, R * D)], sum_hbm.at[pl.ds(lo * D, R * D)])
    pltpu.sync_copy(accm.at[pl.ds(0, R * D)], max_hbm.at[pl.ds(lo * D, R * D)])
    pltpu.sync_copy(cnt.at[pl.ds(0, R)], cnt_hbm.at[pl.ds(lo, R)])


@jax.jit
def _sc_aggregate(x, src, dst):
    mesh = plsc.VectorSubcoreMesh(core_axis_name="c", subcore_axis_name="s")
    f = pl.kernel(
        _sc_body,
        mesh=mesh,
        compiler_params=pltpu.CompilerParams(needs_layout_passes=False),
        out_type=(
            jax.ShapeDtypeStruct((NP * D,), jnp.float32),
            jax.ShapeDtypeStruct((NP,), jnp.float32),
            jax.ShapeDtypeStruct((NP * D,), jnp.float32),
        ),
        scratch_types=[
            pltpu.VMEM((RA * D,), jnp.float32),
            pltpu.VMEM((RA * D,), jnp.float32),
            pltpu.VMEM((RA,), jnp.float32),
            pltpu.VMEM((2 * CS,), jnp.int32),
            pltpu.VMEM((2 * CS,), jnp.int32),
            pltpu.VMEM((PCAP,), jnp.int32),
            pltpu.VMEM((PDCAP,), jnp.int32),
            pltpu.VMEM((PCAP, D), jnp.float32),
            pltpu.SemaphoreType.DMA,
            pltpu.SemaphoreType.DMA,
        ],
    )
    return f(x, src, dst)


def _tc_body(x_ref, s_ref, c_ref, m_ref, w1_ref, w2_ref, w3_ref, b_ref,
             o_ref):
    c = c_ref[...]
    mean = s_ref[...] / jnp.maximum(c, 1.0)
    mx = jnp.where(c > 0.0, m_ref[...], 0.0)
    acc = jnp.dot(x_ref[...], w1_ref[...], preferred_element_type=jnp.float32)
    acc += jnp.dot(mean, w2_ref[...], preferred_element_type=jnp.float32)
    acc += jnp.dot(mx, w3_ref[...], preferred_element_type=jnp.float32)
    o_ref[...] = jnp.maximum(acc + b_ref[...], 0.0)


@jax.jit
def _tc_apply(x, s, c, m, w1, w2, w3, b):
    BN = 400
    grid = (N // BN,)
    row_spec = pl.BlockSpec((BN, D), lambda i: (i, 0))
    full = lambda shape: pl.BlockSpec(shape, lambda i: (0, 0))
    return pl.pallas_call(
        _tc_body,
        grid=grid,
        in_specs=[row_spec,
                  row_spec,
                  pl.BlockSpec((BN, 1), lambda i: (i, 0)),
                  row_spec,
                  full((D, D)), full((D, D)), full((D, D)),
                  full((1, D))],
        out_specs=row_spec,
        out_shape=jax.ShapeDtypeStruct((N, D), jnp.float32),
    )(x, s, c, m, w1, w2, w3, b)


def kernel(x, edge_index, W, b):
    src = edge_index[0]
    dst = edge_index[1]
    s_flat, cnt, m_flat = _sc_aggregate(x, src, dst)
    s = s_flat.reshape(NP, D)[:N]
    m = m_flat.reshape(NP, D)[:N]
    c = cnt[:N, None]
    w1 = W[:, :D].T
    w2 = W[:, D:2 * D].T
    w3 = W[:, 2 * D:].T
    return _tc_apply(x, s, c, m, w1, w2, w3, b[None, :])


# speedup vs baseline: 3.1134x; 1.0042x over previous
import functools

import jax
import jax.numpy as jnp
from jax import lax
from jax.experimental import pallas as pl
from jax.experimental.pallas import tpu as pltpu
from jax.experimental.pallas import tpu_sc as plsc

N = 10000
E = 320000
D = 128

NW = 32
R = 320
RA = R + 16
DUMP = R
NP = NW * R
CS = 3200
NCHUNK = E // CS
BG = 8
NBATCH = CS // (16 * BG)
PCAP = 128
PDCAP = 144
LANES = 16
DC = D // LANES

NEG = float("-inf")


def _sc_body(x_hbm, src_hbm, dst_hbm, sum_hbm, cnt_hbm, max_hbm,
             accs, accm, cnt, dstb, srcb, psrc, pdst, rows, sem, sem2, sem3):
    wid = lax.axis_index("s") * 2 + lax.axis_index("c")
    lo = wid * R
    hi = lo + R

    zs = jnp.zeros((LANES,), jnp.float32)
    negs = jnp.full((LANES,), NEG, jnp.float32)
    zi = jnp.zeros((LANES,), jnp.int32)

    def init_acc(i, _):
        accs[pl.ds(i * LANES, LANES)] = zs
        accm[pl.ds(i * LANES, LANES)] = negs
        return 0
    lax.fori_loop(0, RA * DC, init_acc, 0)

    def init_cnt(i, _):
        cnt[pl.ds(i * LANES, LANES)] = zs
        return 0
    lax.fori_loop(0, RA // LANES, init_cnt, 0)

    def init_pend(i, _):
        pdst[pl.ds(i * LANES, LANES)] = zi
        return 0
    lax.fori_loop(0, PDCAP // LANES, init_pend, 0)

    def init_psrc(i, _):
        psrc[pl.ds(i * LANES, LANES)] = zi
        return 0
    lax.fori_loop(0, PCAP // LANES, init_psrc, 0)

    lanes_iota = lax.iota(jnp.int32, LANES)
    one = jnp.float32(1.0)
    zero = jnp.float32(0.0)

    def do_flush(pos):
        cp1 = pltpu.async_copy(x_hbm.at[psrc.at[pl.ds(0, PCAP // 2)]],
                               rows.at[pl.ds(0, PCAP // 2)], sem)
        cp2 = pltpu.async_copy(x_hbm.at[psrc.at[pl.ds(PCAP // 2, PCAP // 2)]],
                               rows.at[pl.ds(PCAP // 2, PCAP // 2)], sem3)
        blk = pos >> 4
        rem = pos & (LANES - 1)

        @pl.when(rem > 0)
        def _():
            vdt = pdst[pl.ds(blk * LANES, LANES)]
            pdst[pl.ds(blk * LANES, LANES)] = jnp.where(
                lanes_iota >= rem, jnp.int32(DUMP), vdt)

        nb = blk + jnp.where(rem > 0, 1, 0)

        def acc_grp(g, _):
            base = g * LANES
            dv = pdst[pl.ds(base, LANES)]
            for i in range(LANES):
                d = dv[i]
                off = d * D
                e = base + i
                for c in range(DC):
                    r = rows[e, pl.ds(c * LANES, LANES)]
                    plsc.addupdate(accs.at[pl.ds(off + c * LANES, LANES)], r)
                    m = accm[pl.ds(off + c * LANES, LANES)]
                    accm[pl.ds(off + c * LANES, LANES)] = jnp.maximum(m, r)
                onehot = jnp.where(lanes_iota == (d & (LANES - 1)), one, zero)
                plsc.addupdate(cnt.at[pl.ds((d >> 4) * LANES, LANES)], onehot)
            return 0
        nb1 = jnp.minimum(nb, PCAP // 2 // LANES)
        cp1.wait()
        lax.fori_loop(0, nb1, acc_grp, 0)
        cp2.wait()
        lax.fori_loop(nb1, nb, acc_grp, 0)

    one_i = jnp.int32(1)
    zero_i = jnp.int32(0)

    def chunk_body(ci, pos):
        hoff = (ci & 1) * CS
        pltpu.make_async_copy(dst_hbm.at[pl.ds(ci * CS, CS)],
                              dstb.at[pl.ds(hoff, CS)], sem2).wait()
        pltpu.make_async_copy(src_hbm.at[pl.ds(ci * CS, CS)],
                              srcb.at[pl.ds(hoff, CS)], sem2).wait()

        @pl.when(ci + 1 < NCHUNK)
        def _():
            nhoff = CS - hoff
            pltpu.async_copy(dst_hbm.at[pl.ds((ci + 1) * CS, CS)],
                             dstb.at[pl.ds(nhoff, CS)], sem2)
            pltpu.async_copy(src_hbm.at[pl.ds((ci + 1) * CS, CS)],
                             srcb.at[pl.ds(nhoff, CS)], sem2)

        def scan_batch(bi, pos):
            base = hoff + bi * (BG * LANES)
            vds = [dstb[pl.ds(base + k * LANES, LANES)] for k in range(BG)]
            vss = [srcb[pl.ds(base + k * LANES, LANES)] for k in range(BG)]
            msks = [(vd >= lo) & (vd < hi) for vd in vds]
            cums = [plsc.cumsum(jnp.where(m, one_i, zero_i)) for m in msks]
            cnts = [c[LANES - 1] for c in cums]
            total = cnts[0]
            for k in range(1, BG):
                total = total + cnts[k]

            @pl.when(pos + total > PCAP)
            def _():
                do_flush(pos)

            b = jnp.where(pos + total > PCAP, 0, pos)
            newpos = b + total
            for k in range(BG):
                idxv = b + cums[k] - 1
                plsc.store_scatter(psrc, [idxv], vss[k], mask=msks[k])
                plsc.store_scatter(pdst, [idxv], vds[k] - lo, mask=msks[k])
                b = b + cnts[k]
            return newpos

        return lax.fori_loop(0, NBATCH, scan_batch, pos)

    pltpu.async_copy(dst_hbm.at[pl.ds(0, CS)], dstb.at[pl.ds(0, CS)], sem2)
    pltpu.async_copy(src_hbm.at[pl.ds(0, CS)], srcb.at[pl.ds(0, CS)], sem2)
    pos = lax.fori_loop(0, NCHUNK, chunk_body, jnp.int32(0))

    @pl.when(pos > 0)
    def _():
        do_flush(pos)

    pltpu.sync_copy(accs.at[pl.ds(0, R * D)], sum_hbm.at[pl.ds(lo * D, R * D)])
    pltpu.sync_copy(accm.at[pl.ds(0, R * D)], max_hbm.at[pl.ds(lo * D, R * D)])
    pltpu.sync_copy(cnt.at[pl.ds(0, R)], cnt_hbm.at[pl.ds(lo, R)])


@jax.jit
def _sc_aggregate(x, src, dst):
    mesh = plsc.VectorSubcoreMesh(core_axis_name="c", subcore_axis_name="s")
    f = pl.kernel(
        _sc_body,
        mesh=mesh,
        compiler_params=pltpu.CompilerParams(needs_layout_passes=False),
        out_type=(
            jax.ShapeDtypeStruct((NP * D,), jnp.float32),
            jax.ShapeDtypeStruct((NP,), jnp.float32),
            jax.ShapeDtypeStruct((NP * D,), jnp.float32),
        ),
        scratch_types=[
            pltpu.VMEM((RA * D,), jnp.float32),
            pltpu.VMEM((RA * D,), jnp.float32),
            pltpu.VMEM((RA,), jnp.float32),
            pltpu.VMEM((2 * CS,), jnp.int32),
            pltpu.VMEM((2 * CS,), jnp.int32),
            pltpu.VMEM((PCAP,), jnp.int32),
            pltpu.VMEM((PDCAP,), jnp.int32),
            pltpu.VMEM((PCAP, D), jnp.float32),
            pltpu.SemaphoreType.DMA,
            pltpu.SemaphoreType.DMA,
            pltpu.SemaphoreType.DMA,
        ],
    )
    return f(x, src, dst)


def _tc_body(x_ref, s_ref, c_ref, m_ref, w1_ref, w2_ref, w3_ref, b_ref,
             o_ref):
    c = c_ref[...]
    mean = s_ref[...] / jnp.maximum(c, 1.0)
    mx = jnp.where(c > 0.0, m_ref[...], 0.0)
    acc = jnp.dot(x_ref[...], w1_ref[...], preferred_element_type=jnp.float32)
    acc += jnp.dot(mean, w2_ref[...], preferred_element_type=jnp.float32)
    acc += jnp.dot(mx, w3_ref[...], preferred_element_type=jnp.float32)
    o_ref[...] = jnp.maximum(acc + b_ref[...], 0.0)


@jax.jit
def _tc_apply(x, s, c, m, w1, w2, w3, b):
    BN = 400
    grid = (N // BN,)
    row_spec = pl.BlockSpec((BN, D), lambda i: (i, 0))
    full = lambda shape: pl.BlockSpec(shape, lambda i: (0, 0))
    return pl.pallas_call(
        _tc_body,
        grid=grid,
        in_specs=[row_spec,
                  row_spec,
                  pl.BlockSpec((BN, 1), lambda i: (i, 0)),
                  row_spec,
                  full((D, D)), full((D, D)), full((D, D)),
                  full((1, D))],
        out_specs=row_spec,
        out_shape=jax.ShapeDtypeStruct((N, D), jnp.float32),
    )(x, s, c, m, w1, w2, w3, b)


def kernel(x, edge_index, W, b):
    src = edge_index[0]
    dst = edge_index[1]
    s_flat, cnt, m_flat = _sc_aggregate(x, src, dst)
    s = s_flat.reshape(NP, D)[:N]
    m = m_flat.reshape(NP, D)[:N]
    c = cnt[:N, None]
    w1 = W[:, :D].T
    w2 = W[:, D:2 * D].T
    w3 = W[:, 2 * D:].T
    return _tc_apply(x, s, c, m, w1, w2, w3, b[None, :])
